# baseline (speedup 1.0000x reference)
import functools

import jax
import jax.numpy as jnp
from jax import lax
from jax.experimental import pallas as pl
from jax.experimental.pallas import tpu as pltpu
from jax.experimental.pallas import tpu_sc as plsc

NC = 2
NS = 16
NW = NC * NS
NBUF = 8
SLAG = 3


def _emb_call(B0, B1, D, dtype):
    n = B0 // NW
    mesh = plsc.VectorSubcoreMesh(
        core_axis_name="c", subcore_axis_name="s",
        num_cores=NC, num_subcores=NS,
    )

    @functools.partial(
        pl.kernel,
        mesh=mesh,
        compiler_params=pltpu.CompilerParams(use_tc_tiling_on_sc=False),
        out_type=jax.ShapeDtypeStruct((B0, B1, D), dtype),
        scratch_types=[
            pltpu.VMEM((n, B1), jnp.int32),
            pltpu.VMEM((NBUF, B1, D), dtype),
            [pltpu.SemaphoreType.DMA] * NBUF,
            [pltpu.SemaphoreType.DMA] * NBUF,
        ],
    )
    def emb(idx_hbm, tbl_hbm, out_hbm, idx_v, rows_v, gsems, ssems):
        wid = lax.axis_index("s") * NC + lax.axis_index("c")
        rbase = wid * n
        pltpu.sync_copy(idx_hbm.at[pl.ds(rbase, n)], idx_v)
        bufs = [rows_v.at[b] for b in range(NBUF)]

        def start_gather(j, b):
            pltpu.async_copy(tbl_hbm.at[idx_v.at[j]], bufs[b], gsems[b])

        def wait_gather(j, b):
            pltpu.make_async_copy(
                tbl_hbm.at[idx_v.at[j]], bufs[b], gsems[b]).wait()

        def start_write(j, b):
            pltpu.async_copy(bufs[b], out_hbm.at[rbase + j], ssems[b])

        def wait_write(j, b):
            pltpu.make_async_copy(
                bufs[b], out_hbm.at[rbase + j], ssems[b]).wait()

        def chunk(j, b, do_prev):
            wait_gather(j, b)
            start_write(j, b)
            if do_prev:
                b2 = (b - SLAG) % NBUF
                j2 = j - SLAG
                wait_write(j2, b2)
                start_gather(j2 + NBUF, b2)

        G = n // NBUF
        for m in range(NBUF):
            start_gather(m, m)
        for b in range(NBUF):
            chunk(b, b, b >= SLAG)

        def outer(g, carry):
            for b in range(NBUF):
                chunk(g * NBUF + b, b, True)
            return carry

        lax.fori_loop(1, G - 1, outer, 0)
        for b in range(NBUF):
            chunk((G - 1) * NBUF + b, b, b < SLAG)
        for t in range(NBUF):
            j2 = n - NBUF + t
            wait_write(j2, j2 % NBUF)

    return emb


def kernel(x, weight):
    B0, B1 = x.shape
    V, D = weight.shape
    assert B0 % NW == 0 and (B0 // NW) % NBUF == 0 and B0 // (NW * NBUF) >= 2
    idx = x.astype(jnp.int32)
    return _emb_call(B0, B1, D, weight.dtype)(idx, weight)


# speedup vs baseline: 3.1794x; 3.1794x over previous
import functools

import jax
import jax.numpy as jnp
from jax import lax
from jax.experimental import pallas as pl
from jax.experimental.pallas import tpu as pltpu
from jax.experimental.pallas import tpu_sc as plsc

NC = 2
NS = 16
NW = NC * NS
NBUF = 5
SLAG = 2


def _emb_call(B0, B1, D, dtype):
    CB = B0 // NW
    n = B1
    mesh = plsc.VectorSubcoreMesh(
        core_axis_name="c", subcore_axis_name="s",
        num_cores=NC, num_subcores=NS,
    )

    @functools.partial(
        pl.kernel,
        mesh=mesh,
        out_type=jax.ShapeDtypeStruct((B1, B0, D), dtype),
        scratch_types=[
            pltpu.VMEM((B1, CB), jnp.int32),
            pltpu.VMEM((NBUF, CB, D), dtype),
            [pltpu.SemaphoreType.DMA] * NBUF,
            [pltpu.SemaphoreType.DMA] * NBUF,
        ],
    )
    def emb(idx_hbm, tbl_hbm, out_hbm, idx_v, rows_v, gsems, ssems):
        wid = lax.axis_index("s") * NC + lax.axis_index("c")
        cbase = wid * CB
        pltpu.sync_copy(idx_hbm.at[:, pl.ds(cbase, CB)], idx_v)
        bufs = [rows_v.at[b] for b in range(NBUF)]

        def start_gather(j, b):
            pltpu.async_copy(tbl_hbm.at[idx_v.at[j]], bufs[b], gsems[b])

        def wait_gather(j, b):
            pltpu.make_async_copy(
                tbl_hbm.at[idx_v.at[j]], bufs[b], gsems[b]).wait()

        def start_write(j, b):
            pltpu.async_copy(
                bufs[b], out_hbm.at[j, pl.ds(cbase, CB)], ssems[b])

        def wait_write(j, b):
            pltpu.make_async_copy(
                bufs[b], out_hbm.at[j, pl.ds(cbase, CB)], ssems[b]).wait()

        def chunk(j, b, do_prev):
            wait_gather(j, b)
            start_write(j, b)
            if do_prev:
                b2 = (b - SLAG) % NBUF
                j2 = j - SLAG
                wait_write(j2, b2)
                start_gather(j2 + NBUF, b2)

        G = n // NBUF
        for m in range(NBUF):
            start_gather(m, m)
        for b in range(NBUF):
            chunk(b, b, b >= SLAG)

        def outer(g, carry):
            for b in range(NBUF):
                chunk(g * NBUF + b, b, True)
            return carry

        lax.fori_loop(1, G - 1, outer, 0)
        for b in range(NBUF):
            chunk((G - 1) * NBUF + b, b, b < SLAG)
        for t in range(NBUF):
            j2 = n - NBUF + t
            wait_write(j2, j2 % NBUF)

    return emb


def kernel(x, weight):
    B0, B1 = x.shape
    V, D = weight.shape
    assert B0 % (NW * 8) == 0 and B1 % NBUF == 0 and B1 // NBUF >= 2
    xt = jnp.transpose(x).astype(jnp.int32)
    out = _emb_call(B0, B1, D, weight.dtype)(xt, weight)
    return jnp.transpose(out, (1, 0, 2))


# speedup vs baseline: 3.2032x; 1.0075x over previous
import functools

import jax
import jax.numpy as jnp
from jax import lax
from jax.experimental import pallas as pl
from jax.experimental.pallas import tpu as pltpu
from jax.experimental.pallas import tpu_sc as plsc

NC = 2
NS = 16
NW = NC * NS
NBUF = 10
SLAG = 4
NSUB = 2


def _emb_call(B0, B1, D, dtype):
    CB = B0 // NW
    CW = CB // NSUB
    n = B1 * NSUB
    mesh = plsc.VectorSubcoreMesh(
        core_axis_name="c", subcore_axis_name="s",
        num_cores=NC, num_subcores=NS,
    )

    @functools.partial(
        pl.kernel,
        mesh=mesh,
        out_type=jax.ShapeDtypeStruct((B1, B0, D), dtype),
        scratch_types=[
            pltpu.VMEM((B1, CB), jnp.int32),
            pltpu.VMEM((NBUF, CW, D), dtype),
            [pltpu.SemaphoreType.DMA] * NBUF,
            [pltpu.SemaphoreType.DMA] * NBUF,
        ],
    )
    def emb(idx_hbm, tbl_hbm, out_hbm, idx_v, rows_v, gsems, ssems):
        wid = lax.axis_index("s") * NC + lax.axis_index("c")
        cbase = wid * CB
        pltpu.sync_copy(idx_hbm.at[:, pl.ds(cbase, CB)], idx_v)
        bufs = [rows_v.at[b] for b in range(NBUF)]

        def _islice(j):
            b1 = j // NSUB
            s = j - b1 * NSUB
            return idx_v.at[b1, pl.ds(s * CW, CW)]

        def _oslice(j):
            b1 = j // NSUB
            s = j - b1 * NSUB
            return out_hbm.at[b1, pl.ds(cbase + s * CW, CW)]

        def start_gather(j, b):
            pltpu.async_copy(tbl_hbm.at[_islice(j)], bufs[b], gsems[b])

        def wait_gather(j, b):
            pltpu.make_async_copy(
                tbl_hbm.at[_islice(j)], bufs[b], gsems[b]).wait()

        def start_write(j, b):
            pltpu.async_copy(bufs[b], _oslice(j), ssems[b])

        def wait_write(j, b):
            pltpu.make_async_copy(bufs[b], _oslice(j), ssems[b]).wait()

        def chunk(j, b, do_prev):
            wait_gather(j, b)
            start_write(j, b)
            if do_prev:
                b2 = (b - SLAG) % NBUF
                j2 = j - SLAG
                wait_write(j2, b2)
                start_gather(j2 + NBUF, b2)

        G = n // NBUF
        for m in range(NBUF):
            start_gather(m, m)
        for b in range(NBUF):
            chunk(b, b, b >= SLAG)

        def outer(g, carry):
            for b in range(NBUF):
                chunk(g * NBUF + b, b, True)
            return carry

        lax.fori_loop(1, G - 1, outer, 0)
        for b in range(NBUF):
            chunk((G - 1) * NBUF + b, b, b < SLAG)
        for t in range(NBUF):
            j2 = n - NBUF + t
            wait_write(j2, j2 % NBUF)

    return emb


def kernel(x, weight):
    B0, B1 = x.shape
    V, D = weight.shape
    n_ = B1 * NSUB
    assert B0 % (NW * NSUB * 8) == 0 and n_ % NBUF == 0 and n_ // NBUF >= 2
    xt = jnp.transpose(x).astype(jnp.int32)
    out = _emb_call(B0, B1, D, weight.dtype)(xt, weight)
    return jnp.transpose(out, (1, 0, 2))


# speedup vs baseline: 3.2085x; 1.0017x over previous
import functools

import jax
import jax.numpy as jnp
from jax import lax
from jax.experimental import pallas as pl
from jax.experimental.pallas import tpu as pltpu
from jax.experimental.pallas import tpu_sc as plsc

NC = 2
NS = 16
NW = NC * NS
NBUF = 10
SLAG = 4
NSUB = 2


def _emb_call(B0, B1, D, dtype):
    CB = B0 // NW
    CW = CB // NSUB
    n = B1 * NSUB
    mesh = plsc.VectorSubcoreMesh(
        core_axis_name="c", subcore_axis_name="s",
        num_cores=NC, num_subcores=NS,
    )

    @functools.partial(
        pl.kernel,
        mesh=mesh,
        compiler_params=pltpu.CompilerParams(skip_device_barrier=True),
        out_type=jax.ShapeDtypeStruct((B1, B0, D), dtype),
        scratch_types=[
            pltpu.VMEM((B1, CB), jnp.int32),
            pltpu.VMEM((NBUF, CW, D), dtype),
            [pltpu.SemaphoreType.DMA] * NBUF,
            [pltpu.SemaphoreType.DMA] * NBUF,
        ],
    )
    def emb(idx_hbm, tbl_hbm, out_hbm, idx_v, rows_v, gsems, ssems):
        wid = lax.axis_index("s") * NC + lax.axis_index("c")
        cbase = wid * CB
        pltpu.sync_copy(idx_hbm.at[:, pl.ds(cbase, CB)], idx_v)
        bufs = [rows_v.at[b] for b in range(NBUF)]

        def _islice(j):
            b1 = j // NSUB
            s = j - b1 * NSUB
            return idx_v.at[b1, pl.ds(s * CW, CW)]

        def _oslice(j):
            b1 = j // NSUB
            s = j - b1 * NSUB
            return out_hbm.at[b1, pl.ds(cbase + s * CW, CW)]

        def start_gather(j, b):
            pltpu.async_copy(tbl_hbm.at[_islice(j)], bufs[b], gsems[b])

        def wait_gather(j, b):
            pltpu.make_async_copy(
                tbl_hbm.at[_islice(j)], bufs[b], gsems[b]).wait()

        def start_write(j, b):
            pltpu.async_copy(bufs[b], _oslice(j), ssems[b])

        def wait_write(j, b):
            pltpu.make_async_copy(bufs[b], _oslice(j), ssems[b]).wait()

        def chunk(j, b, do_prev):
            wait_gather(j, b)
            start_write(j, b)
            if do_prev:
                b2 = (b - SLAG) % NBUF
                j2 = j - SLAG
                wait_write(j2, b2)
                start_gather(j2 + NBUF, b2)

        G = n // NBUF
        for m in range(NBUF):
            start_gather(m, m)
        for b in range(NBUF):
            chunk(b, b, b >= SLAG)

        def outer(g, carry):
            for b in range(NBUF):
                chunk(g * NBUF + b, b, True)
            return carry

        lax.fori_loop(1, G - 1, outer, 0)
        for b in range(NBUF):
            chunk((G - 1) * NBUF + b, b, b < SLAG)
        for t in range(NBUF):
            j2 = n - NBUF + t
            wait_write(j2, j2 % NBUF)

    return emb


def kernel(x, weight):
    B0, B1 = x.shape
    V, D = weight.shape
    n_ = B1 * NSUB
    assert B0 % (NW * NSUB * 8) == 0 and n_ % NBUF == 0 and n_ // NBUF >= 2
    xt = jnp.transpose(x).astype(jnp.int32)
    out = _emb_call(B0, B1, D, weight.dtype)(xt, weight)
    return jnp.transpose(out, (1, 0, 2))
